# Initial kernel scaffold; baseline (speedup 1.0000x reference)
import jax
import jax.numpy as jnp
from jax.experimental import pallas as pl


def kernel(x, table):
    raise NotImplementedError("write your pallas kernel here")


# speedup vs baseline: 1.4654x; 1.4654x over previous
import functools
import math

import jax
import jax.numpy as jnp
import numpy as np
from jax import lax
from jax.experimental import pallas as pl
from jax.experimental.pallas import tpu as pltpu
from jax.experimental.pallas import tpu_sc as plsc

_MAX_LEN = 200


def _pe_table(max_len, d_model):
    position = np.arange(0, max_len, dtype=np.float32)[:, None]
    div_term = np.exp(
        np.arange(0, d_model, 2, dtype=np.float32) * (-math.log(10000.0) / d_model)
    )
    pe = np.zeros((max_len, d_model), dtype=np.float32)
    pe[:, 0::2] = np.sin(position * div_term)
    if d_model % 2 == 1:
        pe[:, 1::2] = np.cos(position * div_term[:-1])
    else:
        pe[:, 1::2] = np.cos(position * div_term)
    return pe


@functools.partial(jax.jit, static_argnames=("batch", "seq", "d"))
def _embed_pe(table, xf, pe, *, batch, seq, d):
    NC, NS = 2, 16
    NW = NC * NS
    B = batch * seq
    assert batch % NW == 0, batch
    seq_per_w = batch // NW
    ch_seq = 4
    while seq_per_w % ch_seq:
        ch_seq //= 2
    CH = ch_seq * seq
    n_ch = seq_per_w // ch_seq
    assert n_ch % 2 == 0, n_ch
    rows_per_w = seq_per_w * seq
    assert d % 16 == 0, d
    DH = d // 16
    gs = []
    off = 0
    while off < CH:
        n = min(128, CH - off)
        gs.append((off, n))
        off += n

    mesh = plsc.VectorSubcoreMesh(core_axis_name="c", subcore_axis_name="s")

    @functools.partial(
        pl.kernel,
        mesh=mesh,
        out_type=jax.ShapeDtypeStruct((B, d), jnp.float32),
        compiler_params=pltpu.CompilerParams(use_tc_tiling_on_sc=False),
        scratch_types=[
            pltpu.VMEM((CH,), jnp.int32),
            pltpu.VMEM((CH,), jnp.int32),
            pltpu.VMEM((CH, d), jnp.float32),
            pltpu.VMEM((CH, d), jnp.float32),
            pltpu.VMEM((seq, d), jnp.float32),
            pltpu.SemaphoreType.DMA,
        ],
    )
    def k(table_hbm, x_hbm, pe_hbm, out_hbm, idx0, idx1, rows0, rows1, pe_v, gsem):
        idx_b = (idx0, idx1)
        rows_b = (rows0, rows1)
        wid = lax.axis_index("s") * NC + lax.axis_index("c")
        base = wid * rows_per_w

        pltpu.sync_copy(pe_hbm, pe_v)

        def fire(chunk, b):
            cb = base + chunk * CH
            pltpu.sync_copy(x_hbm.at[pl.ds(cb, CH)], idx_b[b])
            for (o, n) in gs:
                pltpu.async_copy(
                    table_hbm.at[idx_b[b].at[pl.ds(o, n)]],
                    rows_b[b].at[pl.ds(o, n)],
                    gsem,
                )

        def drain(b):
            for (o, n) in gs:
                pltpu.make_async_copy(
                    table_hbm.at[idx_b[b].at[pl.ds(o, n)]],
                    rows_b[b].at[pl.ds(o, n)],
                    gsem,
                ).wait()

        def add_pe(b):
            def row_body(r, _):
                for h in range(DH):
                    pv = pe_v[r, pl.ds(h * 16, 16)]
                    for s in range(ch_seq):
                        plsc.addupdate(
                            rows_b[b].at[s * seq + r, pl.ds(h * 16, 16)], pv
                        )
                return 0

            lax.fori_loop(0, seq, row_body, 0)

        fire(0, 0)

        def step2(i, _):
            c0 = i * 2
            for b in (0, 1):
                c = c0 + b

                @pl.when(c + 1 < n_ch)
                def _():
                    fire(c + 1, 1 - b)

                drain(b)
                add_pe(b)
                pltpu.sync_copy(
                    rows_b[b], out_hbm.at[pl.ds(base + c * CH, CH)]
                )
            return 0

        lax.fori_loop(0, n_ch // 2, step2, 0)

    return k(table, xf, pe)


def kernel(x, table):
    batch, seq = x.shape
    _, d = table.shape
    pe = jnp.asarray(_pe_table(_MAX_LEN, d)[:seq])
    out = _embed_pe(table, x.reshape(-1), pe, batch=batch, seq=seq, d=d)
    return out.reshape(batch, seq, d)


# speedup vs baseline: 1.4946x; 1.0199x over previous
import functools
import math

import jax
import jax.numpy as jnp
import numpy as np
from jax import lax
from jax.experimental import pallas as pl
from jax.experimental.pallas import tpu as pltpu
from jax.experimental.pallas import tpu_sc as plsc

_MAX_LEN = 200


def _pe_table(max_len, d_model):
    position = np.arange(0, max_len, dtype=np.float32)[:, None]
    div_term = np.exp(
        np.arange(0, d_model, 2, dtype=np.float32) * (-math.log(10000.0) / d_model)
    )
    pe = np.zeros((max_len, d_model), dtype=np.float32)
    pe[:, 0::2] = np.sin(position * div_term)
    if d_model % 2 == 1:
        pe[:, 1::2] = np.cos(position * div_term[:-1])
    else:
        pe[:, 1::2] = np.cos(position * div_term)
    return pe


@functools.partial(jax.jit, static_argnames=("batch", "seq", "d"))
def _embed_pe(table, x, pe, *, batch, seq, d):
    NC, NS = 2, 16
    NW = NC * NS
    assert batch % NW == 0, batch
    seq_per_w = batch // NW
    ch_seq = 4
    while seq_per_w % ch_seq:
        ch_seq //= 2
    n_ch = seq_per_w // ch_seq
    assert n_ch % 2 == 0, n_ch
    assert d % 16 == 0, d
    DH = d // 16
    gs = []
    off = 0
    while off < seq:
        n = min(128, seq - off)
        gs.append((off, n))
        off += n

    mesh = plsc.VectorSubcoreMesh(core_axis_name="c", subcore_axis_name="s")

    @functools.partial(
        pl.kernel,
        mesh=mesh,
        out_type=jax.ShapeDtypeStruct((batch, seq, d), jnp.float32),
        compiler_params=pltpu.CompilerParams(use_tc_tiling_on_sc=False),
        scratch_types=[
            pltpu.VMEM((seq_per_w, seq), jnp.int32),
            pltpu.VMEM((ch_seq, seq, d), jnp.float32),
            pltpu.VMEM((ch_seq, seq, d), jnp.float32),
            pltpu.VMEM((seq, d), jnp.float32),
            pltpu.SemaphoreType.DMA,
            pltpu.SemaphoreType.DMA,
        ],
    )
    def k(table_hbm, x_hbm, pe_hbm, out_hbm, idx_v, rows0, rows1, pe_v, gsem, ssem):
        rows_b = (rows0, rows1)
        wid = lax.axis_index("s") * NC + lax.axis_index("c")
        sbase = wid * seq_per_w

        pltpu.sync_copy(pe_hbm, pe_v)
        pltpu.sync_copy(x_hbm.at[pl.ds(sbase, seq_per_w)], idx_v)

        def fire(chunk, b):
            for s in range(ch_seq):
                si = chunk * ch_seq + s
                for (o, n) in gs:
                    pltpu.async_copy(
                        table_hbm.at[idx_v.at[si].at[pl.ds(o, n)]],
                        rows_b[b].at[s].at[pl.ds(o, n)],
                        gsem,
                    )

        def drain(chunk, b):
            for s in range(ch_seq):
                si = chunk * ch_seq + s
                for (o, n) in gs:
                    pltpu.make_async_copy(
                        table_hbm.at[idx_v.at[si].at[pl.ds(o, n)]],
                        rows_b[b].at[s].at[pl.ds(o, n)],
                        gsem,
                    ).wait()

        def store(c, b):
            pltpu.async_copy(
                rows_b[b], out_hbm.at[pl.ds(sbase + c * ch_seq, ch_seq)], ssem
            )

        def wait_store(c, b):
            pltpu.make_async_copy(
                rows_b[b], out_hbm.at[pl.ds(sbase + c * ch_seq, ch_seq)], ssem
            ).wait()

        def add_pe(b):
            def _row(r, _):
                for h in range(DH):
                    pv = pe_v[r, pl.ds(h * 16, 16)]
                    for s in range(ch_seq):
                        plsc.addupdate(rows_b[b].at[s, r, pl.ds(h * 16, 16)], pv)
                return 0

            lax.fori_loop(0, seq, _row, 0)

        fire(0, 0)

        def step2(i, _):
            c0 = i * 2
            for b in (0, 1):
                c = c0 + b

                @pl.when(c + 1 < n_ch)
                def _():
                    @pl.when(c >= 1)
                    def _():
                        wait_store(c - 1, 1 - b)

                    fire(c + 1, 1 - b)

                drain(c, b)
                add_pe(b)
                store(c, b)
            return 0

        lax.fori_loop(0, n_ch // 2, step2, 0)
        wait_store(n_ch - 2, (n_ch - 2) % 2)
        wait_store(n_ch - 1, (n_ch - 1) % 2)

    return k(table, x, pe)


def kernel(x, table):
    batch, seq = x.shape
    _, d = table.shape
    pe = jnp.asarray(_pe_table(_MAX_LEN, d)[:seq])
    return _embed_pe(table, x, pe, batch=batch, seq=seq, d=d)
